# baseline (speedup 1.0000x reference)
import functools

import jax
import jax.numpy as jnp
from jax import lax
from jax.experimental import pallas as pl
from jax.experimental.pallas import tpu as pltpu
from jax.experimental.pallas import tpu_sc as plsc

B = 4096
N_DENSE = 13
N_SPARSE = 26
VOCAB = 1000
EDIM = 128
EPS = 1e-3
ROWS = B * N_SPARSE
NW = 32
RPW = ROWS // NW
CH = 128
NCHUNK = RPW // CH
NBUF = 2
SPLITS = (1024, 1536, 1536)


@functools.cache
def _make_sc_gather(nb):
    nchunk = N_SPARSE * (nb // CH)
    lo = nchunk // NW
    hi = lo + (1 if nchunk % NW else 0)
    nhi = nchunk - lo * NW
    rows_pad = -(-hi // 8) * 8
    mesh = plsc.VectorSubcoreMesh(core_axis_name="c", subcore_axis_name="s")
    return pl.kernel(
        functools.partial(_sc_gather_body, nb=nb, lo=lo, hi=hi, nhi=nhi),
        out_type=jax.ShapeDtypeStruct((nb, N_SPARSE * EDIM), jnp.float32),
        mesh=mesh,
        scratch_types=[
            pltpu.VMEM((rows_pad, CH), jnp.int32),
            pltpu.VMEM((CH, EDIM), jnp.float32),
            pltpu.VMEM((CH, EDIM), jnp.float32),
            pltpu.SemaphoreType.DMA,
            pltpu.SemaphoreType.DMA,
        ],
    )


def _sc_gather_body(table_hbm, idx_hbm, out_hbm, idx_v, buf0, buf1, sem0,
                    sem1, *, nb, lo, hi, nhi):
    nbb = nb // CH
    wid = lax.axis_index("s") * 2 + lax.axis_index("c")
    pltpu.sync_copy(idx_hbm.at[wid], idx_v)
    myn = jnp.where(wid < nhi, hi, lo)
    base = jnp.where(wid < nhi, wid * hi, nhi * hi + (wid - nhi) * lo)
    bufs = (buf0, buf1)
    sems = (sem0, sem1)

    def start(q, slot):
        pltpu.async_copy(table_hbm.at[idx_v.at[q]], bufs[slot], sems[slot])

    def wait_wb(q, slot):
        pltpu.make_async_copy(table_hbm.at[idx_v.at[0]], bufs[slot],
                              sems[slot]).wait()
        c = base + q
        field = c // nbb
        brow = (c % nbb) * CH
        pltpu.sync_copy(bufs[slot],
                        out_hbm.at[pl.ds(brow, CH),
                                   pl.ds(field * EDIM, EDIM)])

    for slot in range(NBUF):
        start(slot, slot)

    n_main = (lo // NBUF) * NBUF

    def outer(q0, carry):
        for slot in range(NBUF):
            q = q0 * NBUF + slot
            wait_wb(q, slot)

            @pl.when(q + NBUF < myn)
            def _():
                start(q + NBUF, slot)
        return carry

    lax.fori_loop(0, n_main // NBUF, outer, 0)
    for q in range(n_main, hi):
        @pl.when(myn > q)
        def _():
            wait_wb(q, q % NBUF)


_S = float(1.0 / (1.0 + EPS) ** 0.5)


def _mlp_body(inp_ref, emb_ref, w_ref, W1s_ref, W1d_ref, b1_ref, g1_ref,
              e1_ref, W2_ref, b2_ref, g2_ref, e2_ref, W3_ref, b3_ref,
              g3_ref, e3_ref, Wo_ref, wb_ref, out_ref):
    f32 = jnp.float32
    bf16 = jnp.bfloat16
    inp = inp_ref[...]
    h = jnp.dot(emb_ref[...].astype(bf16), W1s_ref[...],
                preferred_element_type=f32)
    h = h + jnp.dot(inp, W1d_ref[...], preferred_element_type=f32)
    h = jnp.maximum((h + b1_ref[...]) * (g1_ref[...] * _S) + e1_ref[...],
                    0.0)
    h = jnp.dot(h.astype(bf16), W2_ref[...], preferred_element_type=f32)
    h = jnp.maximum((h + b2_ref[...]) * (g2_ref[...] * _S) + e2_ref[...],
                    0.0)
    h = jnp.dot(h.astype(bf16), W3_ref[...], preferred_element_type=f32)
    h = jnp.maximum((h + b3_ref[...]) * (g3_ref[...] * _S) + e3_ref[...],
                    0.0)
    deep = jnp.dot(h.astype(bf16), Wo_ref[...], preferred_element_type=f32)
    wide = jnp.dot(inp, w_ref[...], preferred_element_type=f32)
    out_ref[...] = jax.nn.sigmoid(deep + wide + wb_ref[...])


def _mlp_call(inputs, emb, w2d, W1s, W1d_ext, *rest):
    nb = inputs.shape[0]
    blk = 512
    grid = (nb // blk,)
    full = lambda a: pl.BlockSpec(a.shape, lambda i: (0,) * a.ndim)
    in_specs = [
        pl.BlockSpec((blk, N_DENSE + N_SPARSE), lambda i: (i, 0)),
        pl.BlockSpec((blk, N_SPARSE * EDIM), lambda i: (i, 0)),
        full(w2d), full(W1s), full(W1d_ext),
    ] + [full(a) for a in rest]
    return pl.pallas_call(
        _mlp_body,
        grid=grid,
        in_specs=in_specs,
        out_specs=pl.BlockSpec((blk, 1), lambda i: (i, 0)),
        out_shape=jax.ShapeDtypeStruct((nb, 1), jnp.float32),
    )(inputs, emb, w2d, W1s, W1d_ext, *rest)


def kernel(inputs, tables, w, b, W1, B1, g1, be1, W2, B2, g2, be2, W3, B3,
           g3, be3, Wo, Bo):
    idx = inputs[:, N_DENSE:].astype(jnp.int32)
    flat_idx = (idx + jnp.arange(N_SPARSE, dtype=jnp.int32)[None, :] * VOCAB)
    tables_flat = tables.reshape(N_SPARSE * VOCAB, EDIM)

    W1s = W1[: N_SPARSE * EDIM].astype(jnp.bfloat16)
    W1d_ext = jnp.pad(W1[N_SPARSE * EDIM:], ((0, N_SPARSE), (0, 0)))
    vecs = (B1.reshape(1, -1), g1.reshape(1, -1), be1.reshape(1, -1),
            W2.astype(jnp.bfloat16), B2.reshape(1, -1), g2.reshape(1, -1),
            be2.reshape(1, -1), W3.astype(jnp.bfloat16), B3.reshape(1, -1),
            g3.reshape(1, -1), be3.reshape(1, -1), Wo.astype(jnp.bfloat16),
            (b + Bo).reshape(1, 1))

    outs = []
    off = 0
    for nb in SPLITS:
        nbb = nb // CH
        nchunk = N_SPARSE * nbb
        lo = nchunk // NW
        hi = lo + (1 if nchunk % NW else 0)
        nhi = nchunk - lo * NW
        rows_pad = -(-hi // 8) * 8
        rows = flat_idx[off:off + nb].T.reshape(nchunk, CH)
        if nhi:
            first = rows[: nhi * hi].reshape(nhi, hi, CH)
            first = jnp.pad(first, ((0, 0), (0, rows_pad - hi), (0, 0)))
            last = rows[nhi * hi:].reshape(NW - nhi, lo, CH)
            last = jnp.pad(last, ((0, 0), (0, rows_pad - lo), (0, 0)))
            idx_h = jnp.concatenate([first, last], axis=0)
        else:
            idx_h = rows.reshape(NW, lo, CH)
            idx_h = jnp.pad(idx_h, ((0, 0), (0, rows_pad - lo), (0, 0)))
        emb_h = _make_sc_gather(nb)(tables_flat, idx_h)
        outs.append(_mlp_call(inputs[off:off + nb], emb_h, w,
                              W1s, W1d_ext, *vecs))
        off += nb
    return jnp.concatenate(outs, axis=0) if len(SPLITS) > 1 else outs[0]


# speedup vs baseline: 1.1492x; 1.1492x over previous
import functools

import jax
import jax.numpy as jnp
from jax import lax
from jax.experimental import pallas as pl
from jax.experimental.pallas import tpu as pltpu
from jax.experimental.pallas import tpu_sc as plsc

B = 4096
N_DENSE = 13
N_SPARSE = 26
VOCAB = 1000
EDIM = 128
EPS = 1e-3
ROWS = B * N_SPARSE
NW = 32
RPW = ROWS // NW
CH = 128
NCHUNK = RPW // CH
NBUF = 2
SPLITS = (2048, 2048)


@functools.cache
def _make_sc_gather(nb):
    nchunk = N_SPARSE * (nb // CH)
    lo = nchunk // NW
    hi = lo + (1 if nchunk % NW else 0)
    nhi = nchunk - lo * NW
    rows_pad = -(-hi // 8) * 8
    mesh = plsc.VectorSubcoreMesh(core_axis_name="c", subcore_axis_name="s")
    if nhi == 0:
        return pl.kernel(
            functools.partial(_sc_gather_ring_body, nb=nb, ncw=lo),
            out_type=jax.ShapeDtypeStruct((nb, N_SPARSE * EDIM),
                                          jnp.float32),
            mesh=mesh,
            scratch_types=(
                [pltpu.VMEM((rows_pad, CH), jnp.int32)]
                + [pltpu.VMEM((CH, EDIM), jnp.float32)] * 4
                + [pltpu.SemaphoreType.DMA] * 8
            ),
        )
    return pl.kernel(
        functools.partial(_sc_gather_body, nb=nb, lo=lo, hi=hi, nhi=nhi),
        out_type=jax.ShapeDtypeStruct((nb, N_SPARSE * EDIM), jnp.float32),
        mesh=mesh,
        scratch_types=[
            pltpu.VMEM((rows_pad, CH), jnp.int32),
            pltpu.VMEM((CH, EDIM), jnp.float32),
            pltpu.VMEM((CH, EDIM), jnp.float32),
            pltpu.SemaphoreType.DMA,
            pltpu.SemaphoreType.DMA,
        ],
    )


def _sc_gather_ring_body(table_hbm, idx_hbm, out_hbm, idx_v, b0, b1, b2, b3,
                         g0, g1, g2, g3, w0, w1, w2, w3, *, nb, ncw):
    nbb = nb // CH
    wid = lax.axis_index("s") * 2 + lax.axis_index("c")
    pltpu.sync_copy(idx_hbm.at[wid], idx_v)
    base = wid * ncw
    bufs = (b0, b1, b2, b3)
    gsems = (g0, g1, g2, g3)
    wsems = (w0, w1, w2, w3)

    def chunk_out(q):
        c = base + q
        return out_hbm.at[pl.ds((c % nbb) * CH, CH),
                          pl.ds((c // nbb) * EDIM, EDIM)]

    def start_gather(q, slot):
        pltpu.async_copy(table_hbm.at[idx_v.at[q]], bufs[slot], gsems[slot])

    def wait_gather(slot):
        pltpu.make_async_copy(table_hbm.at[idx_v.at[0]], bufs[slot],
                              gsems[slot]).wait()

    def start_wb(q, slot):
        pltpu.async_copy(bufs[slot], chunk_out(q), wsems[slot])

    def wait_wb(q, slot):
        pltpu.make_async_copy(bufs[slot], chunk_out(q), wsems[slot]).wait()

    for slot in range(4):
        start_gather(slot, slot)
    for q in range(ncw):
        slot = q % 4
        if 2 <= q and q + 2 < ncw:
            wait_wb(q - 2, (q + 2) % 4)
            start_gather(q + 2, (q + 2) % 4)
        wait_gather(slot)
        start_wb(q, slot)
    for q in range(max(ncw - 4, 0), ncw):
        wait_wb(q, q % 4)


def _sc_gather_body(table_hbm, idx_hbm, out_hbm, idx_v, buf0, buf1, sem0,
                    sem1, *, nb, lo, hi, nhi):
    nbb = nb // CH
    wid = lax.axis_index("s") * 2 + lax.axis_index("c")
    pltpu.sync_copy(idx_hbm.at[wid], idx_v)
    myn = jnp.where(wid < nhi, hi, lo)
    base = jnp.where(wid < nhi, wid * hi, nhi * hi + (wid - nhi) * lo)
    bufs = (buf0, buf1)
    sems = (sem0, sem1)

    def start(q, slot):
        pltpu.async_copy(table_hbm.at[idx_v.at[q]], bufs[slot], sems[slot])

    def wait_wb(q, slot):
        pltpu.make_async_copy(table_hbm.at[idx_v.at[0]], bufs[slot],
                              sems[slot]).wait()
        c = base + q
        field = c // nbb
        brow = (c % nbb) * CH
        pltpu.sync_copy(bufs[slot],
                        out_hbm.at[pl.ds(brow, CH),
                                   pl.ds(field * EDIM, EDIM)])

    for slot in range(NBUF):
        start(slot, slot)

    n_main = (lo // NBUF) * NBUF

    def outer(q0, carry):
        for slot in range(NBUF):
            q = q0 * NBUF + slot
            wait_wb(q, slot)

            @pl.when(q + NBUF < myn)
            def _():
                start(q + NBUF, slot)
        return carry

    lax.fori_loop(0, n_main // NBUF, outer, 0)
    for q in range(n_main, hi):
        @pl.when(myn > q)
        def _():
            wait_wb(q, q % NBUF)


_S = float(1.0 / (1.0 + EPS) ** 0.5)


def _mlp_body(inp_ref, emb_ref, w_ref, W1s_ref, W1d_ref, b1_ref, g1_ref,
              e1_ref, W2_ref, b2_ref, g2_ref, e2_ref, W3_ref, b3_ref,
              g3_ref, e3_ref, Wo_ref, wb_ref, out_ref):
    f32 = jnp.float32
    bf16 = jnp.bfloat16
    inp = inp_ref[...]
    h = jnp.dot(emb_ref[...].astype(bf16), W1s_ref[...],
                preferred_element_type=f32)
    h = h + jnp.dot(inp, W1d_ref[...], preferred_element_type=f32)
    h = jnp.maximum((h + b1_ref[...]) * (g1_ref[...] * _S) + e1_ref[...],
                    0.0)
    h = jnp.dot(h.astype(bf16), W2_ref[...], preferred_element_type=f32)
    h = jnp.maximum((h + b2_ref[...]) * (g2_ref[...] * _S) + e2_ref[...],
                    0.0)
    h = jnp.dot(h.astype(bf16), W3_ref[...], preferred_element_type=f32)
    h = jnp.maximum((h + b3_ref[...]) * (g3_ref[...] * _S) + e3_ref[...],
                    0.0)
    deep = jnp.dot(h.astype(bf16), Wo_ref[...], preferred_element_type=f32)
    wide = jnp.dot(inp, w_ref[...], preferred_element_type=f32)
    out_ref[...] = jax.nn.sigmoid(deep + wide + wb_ref[...])


def _mlp_call(inputs, emb, w2d, W1s, W1d_ext, *rest):
    nb = inputs.shape[0]
    blk = 512
    grid = (nb // blk,)
    full = lambda a: pl.BlockSpec(a.shape, lambda i: (0,) * a.ndim)
    in_specs = [
        pl.BlockSpec((blk, N_DENSE + N_SPARSE), lambda i: (i, 0)),
        pl.BlockSpec((blk, N_SPARSE * EDIM), lambda i: (i, 0)),
        full(w2d), full(W1s), full(W1d_ext),
    ] + [full(a) for a in rest]
    return pl.pallas_call(
        _mlp_body,
        grid=grid,
        in_specs=in_specs,
        out_specs=pl.BlockSpec((blk, 1), lambda i: (i, 0)),
        out_shape=jax.ShapeDtypeStruct((nb, 1), jnp.float32),
    )(inputs, emb, w2d, W1s, W1d_ext, *rest)


def kernel(inputs, tables, w, b, W1, B1, g1, be1, W2, B2, g2, be2, W3, B3,
           g3, be3, Wo, Bo):
    idx = inputs[:, N_DENSE:].astype(jnp.int32)
    flat_idx = (idx + jnp.arange(N_SPARSE, dtype=jnp.int32)[None, :] * VOCAB)
    tables_flat = tables.reshape(N_SPARSE * VOCAB, EDIM)

    W1s = W1[: N_SPARSE * EDIM].astype(jnp.bfloat16)
    W1d_ext = jnp.pad(W1[N_SPARSE * EDIM:], ((0, N_SPARSE), (0, 0)))
    vecs = (B1.reshape(1, -1), g1.reshape(1, -1), be1.reshape(1, -1),
            W2.astype(jnp.bfloat16), B2.reshape(1, -1), g2.reshape(1, -1),
            be2.reshape(1, -1), W3.astype(jnp.bfloat16), B3.reshape(1, -1),
            g3.reshape(1, -1), be3.reshape(1, -1), Wo.astype(jnp.bfloat16),
            (b + Bo).reshape(1, 1))

    outs = []
    off = 0
    for nb in SPLITS:
        nbb = nb // CH
        nchunk = N_SPARSE * nbb
        lo = nchunk // NW
        hi = lo + (1 if nchunk % NW else 0)
        nhi = nchunk - lo * NW
        rows_pad = -(-hi // 8) * 8
        rows = flat_idx[off:off + nb].T.reshape(nchunk, CH)
        if nhi:
            first = rows[: nhi * hi].reshape(nhi, hi, CH)
            first = jnp.pad(first, ((0, 0), (0, rows_pad - hi), (0, 0)))
            last = rows[nhi * hi:].reshape(NW - nhi, lo, CH)
            last = jnp.pad(last, ((0, 0), (0, rows_pad - lo), (0, 0)))
            idx_h = jnp.concatenate([first, last], axis=0)
        else:
            idx_h = rows.reshape(NW, lo, CH)
            idx_h = jnp.pad(idx_h, ((0, 0), (0, rows_pad - lo), (0, 0)))
        emb_h = _make_sc_gather(nb)(tables_flat, idx_h)
        outs.append(_mlp_call(inputs[off:off + nb], emb_h, w,
                              W1s, W1d_ext, *vecs))
        off += nb
    return jnp.concatenate(outs, axis=0) if len(SPLITS) > 1 else outs[0]
